# baseline (speedup 1.0000x reference)
import functools

import jax
import jax.numpy as jnp
from jax import lax
from jax.experimental import pallas as pl
from jax.experimental.pallas import tpu as pltpu
from jax.experimental.pallas import tpu_sc as plsc

_INFO = plsc.get_sparse_core_info()
_NC = _INFO.num_cores
_NS = _INFO.num_subcores
_NW = _NC * _NS
_CHUNK = 128


@functools.lru_cache(maxsize=None)
def _build(n_chunks, chunk, vocab, dim):
    mesh = plsc.VectorSubcoreMesh(core_axis_name="c", subcore_axis_name="s")

    @functools.partial(
        pl.kernel,
        mesh=mesh,
        out_type=jax.ShapeDtypeStruct((_NW, n_chunks, chunk, dim), jnp.float32),
        scratch_types=[
            pltpu.VMEM((n_chunks, chunk), jnp.int32),
            pltpu.VMEM((chunk, dim), jnp.float32),
            pltpu.SemaphoreType.DMA,
        ],
        compiler_params=pltpu.CompilerParams(use_tc_tiling_on_sc=False),
    )
    def gather_kernel(x_hbm, table_hbm, out_hbm, idx_v, row_v, gsem):
        wid = lax.axis_index("s") * _NC + lax.axis_index("c")
        pltpu.sync_copy(x_hbm.at[wid], idx_v)
        out_w = out_hbm.at[wid]

        def chunk_body(j, carry):
            pltpu.async_copy(table_hbm.at[idx_v.at[j]], row_v, gsem).wait()
            pltpu.sync_copy(row_v, out_w.at[j])
            return carry

        lax.fori_loop(0, n_chunks, chunk_body, 0)

    return gather_kernel


def kernel(x, table):
    bsz, hist = x.shape
    vocab, dim = table.shape
    total = bsz * hist
    assert total % (_NW * _CHUNK) == 0
    n_chunks = total // (_NW * _CHUNK)
    xw = x.reshape(_NW, n_chunks, _CHUNK).astype(jnp.int32)
    out = _build(n_chunks, _CHUNK, vocab, dim)(xw, table)
    return out.reshape(bsz, hist, dim)


# speedup vs baseline: 1.1120x; 1.1120x over previous
import functools

import jax
import jax.numpy as jnp
from jax import lax
from jax.experimental import pallas as pl
from jax.experimental.pallas import tpu as pltpu
from jax.experimental.pallas import tpu_sc as plsc

_INFO = plsc.get_sparse_core_info()
_NC = _INFO.num_cores
_NS = _INFO.num_subcores
_NW = _NC * _NS
_CHUNK = 128
_NBUF = 4


@functools.lru_cache(maxsize=None)
def _build(n_chunks, chunk, vocab, dim):
    assert n_chunks % _NBUF == 0
    n_groups = n_chunks // _NBUF
    mesh = plsc.VectorSubcoreMesh(core_axis_name="c", subcore_axis_name="s")

    @functools.partial(
        pl.kernel,
        mesh=mesh,
        out_type=jax.ShapeDtypeStruct((_NW, n_chunks, chunk, dim), jnp.float32),
        scratch_types=[
            pltpu.VMEM((n_chunks, chunk), jnp.int32),
            pltpu.VMEM((_NBUF, chunk, dim), jnp.float32),
            pltpu.SemaphoreType.DMA((_NBUF,)),
            pltpu.SemaphoreType.DMA((_NBUF,)),
        ],
        compiler_params=pltpu.CompilerParams(use_tc_tiling_on_sc=False),
    )
    def gather_kernel(x_hbm, table_hbm, out_hbm, idx_v, bufs, gsems, wsems):
        wid = lax.axis_index("s") * _NC + lax.axis_index("c")
        pltpu.sync_copy(x_hbm.at[wid], idx_v)
        out_w = out_hbm.at[wid]

        def start_gather(j, b):
            pltpu.make_async_copy(
                table_hbm.at[idx_v.at[j]], bufs.at[b], gsems.at[b]
            ).start()

        def wait_gather(j, b):
            pltpu.make_async_copy(
                table_hbm.at[idx_v.at[j]], bufs.at[b], gsems.at[b]
            ).wait()

        def start_write(j, b):
            pltpu.make_async_copy(bufs.at[b], out_w.at[j], wsems.at[b]).start()

        def wait_write(j, b):
            pltpu.make_async_copy(bufs.at[b], out_w.at[j], wsems.at[b]).wait()

        for b in range(_NBUF):
            start_gather(b, b)

        def group(g, carry):
            base = g * _NBUF
            for b in range(_NBUF):
                wait_gather(base + b, b)
                start_write(base + b, b)
            for b in range(_NBUF):
                wait_write(base + b, b)
                start_gather(base + _NBUF + b, b)
            return carry

        lax.fori_loop(0, n_groups - 1, group, 0)

        base = (n_groups - 1) * _NBUF
        for b in range(_NBUF):
            wait_gather(base + b, b)
            start_write(base + b, b)
        for b in range(_NBUF):
            wait_write(base + b, b)

    return gather_kernel


def kernel(x, table):
    bsz, hist = x.shape
    vocab, dim = table.shape
    total = bsz * hist
    assert total % (_NW * _CHUNK) == 0
    n_chunks = total // (_NW * _CHUNK)
    xw = x.reshape(_NW, n_chunks, _CHUNK).astype(jnp.int32)
    out = _build(n_chunks, _CHUNK, vocab, dim)(xw, table)
    return out.reshape(bsz, hist, dim)
